# Initial kernel scaffold; baseline (speedup 1.0000x reference)
import jax
import jax.numpy as jnp
from jax.experimental import pallas as pl


def kernel(hidden_states, gate_w, w1, w3, w2, shared_gate_up, shared_down):
    raise NotImplementedError("write your pallas kernel here")


# speedup vs baseline: 1.1182x; 1.1182x over previous
import jax
import jax.numpy as jnp
from jax.experimental import pallas as pl
from jax.experimental.pallas import tpu as pltpu

_TOP_K = 8


def _moe_body(x_ref, gate_ref, w1_ref, w3_ref, w2_ref, sgu_ref, sd_ref,
              out_ref, comb_ref):
    e = pl.program_id(0)
    n_e = pl.num_programs(0)
    x = x_ref[...]

    @pl.when(e == 0)
    def _router():
        logits = jnp.dot(x, gate_ref[...].T,
                         preferred_element_type=jnp.float32)
        m = jnp.max(logits, axis=-1, keepdims=True)
        p = jnp.exp(logits - m)
        p = p / jnp.sum(p, axis=-1, keepdims=True)
        lane = jax.lax.broadcasted_iota(jnp.int32, p.shape, 1)
        pm = p
        combw = jnp.zeros_like(p)
        for _ in range(_TOP_K):
            rm = jnp.max(pm, axis=-1, keepdims=True)
            eq = (pm == rm)
            first_idx = jnp.min(jnp.where(eq, lane, p.shape[1]), axis=-1,
                                keepdims=True)
            first = lane == first_idx
            combw = jnp.where(first, p, combw)
            pm = jnp.where(first, -jnp.inf, pm)
        denom = jnp.sum(combw, axis=-1, keepdims=True) + 1e-20
        comb_ref[...] = combw / denom
        out_ref[...] = jnp.zeros_like(out_ref)

    w1 = w1_ref[0]
    w3 = w3_ref[0]
    w2 = w2_ref[0]
    h1 = jnp.dot(x, w1.T, preferred_element_type=jnp.float32)
    h3 = jnp.dot(x, w3.T, preferred_element_type=jnp.float32)
    h = h1 * jax.nn.sigmoid(h1) * h3
    oe = jnp.dot(h, w2.T, preferred_element_type=jnp.float32)
    lane = jax.lax.broadcasted_iota(jnp.int32, comb_ref.shape, 1)
    wcol = jnp.sum(jnp.where(lane == e, comb_ref[...], 0.0), axis=1,
                   keepdims=True)
    out_ref[...] += wcol * oe

    @pl.when(e == n_e - 1)
    def _shared():
        gu = jnp.dot(x, sgu_ref[...].T, preferred_element_type=jnp.float32)
        si = sgu_ref.shape[0] // 2
        g = gu[:, :si]
        u = gu[:, si:]
        act = g * jax.nn.sigmoid(g) * u
        out_ref[...] += jnp.dot(act, sd_ref[...].T,
                                preferred_element_type=jnp.float32)


def kernel(hidden_states, gate_w, w1, w3, w2, shared_gate_up, shared_down):
    b, s, d = hidden_states.shape
    x = hidden_states.reshape(-1, d)
    t = x.shape[0]
    e, ffn, _ = w1.shape
    out = pl.pallas_call(
        _moe_body,
        grid=(e,),
        in_specs=[
            pl.BlockSpec((t, d), lambda i: (0, 0)),
            pl.BlockSpec(gate_w.shape, lambda i: (0, 0)),
            pl.BlockSpec((1, ffn, d), lambda i: (i, 0, 0)),
            pl.BlockSpec((1, ffn, d), lambda i: (i, 0, 0)),
            pl.BlockSpec((1, d, ffn), lambda i: (i, 0, 0)),
            pl.BlockSpec(shared_gate_up.shape, lambda i: (0, 0)),
            pl.BlockSpec(shared_down.shape, lambda i: (0, 0)),
        ],
        out_specs=pl.BlockSpec((t, d), lambda i: (0, 0)),
        out_shape=jax.ShapeDtypeStruct((t, d), jnp.float32),
        scratch_shapes=[pltpu.VMEM((t, e), jnp.float32)],
        compiler_params=pltpu.CompilerParams(
            dimension_semantics=("arbitrary",),
        ),
    )(x, gate_w, w1, w3, w2, shared_gate_up, shared_down)
    return out.reshape(b, s, d)
